# Initial kernel scaffold; baseline (speedup 1.0000x reference)
import jax
import jax.numpy as jnp
from jax.experimental import pallas as pl


def kernel(x, codebooks):
    raise NotImplementedError("write your pallas kernel here")


# speedup vs baseline: 4.0604x; 4.0604x over previous
import jax
import jax.numpy as jnp
from jax.experimental import pallas as pl

B, C, F, T = 16, 2, 256, 512
G = 4
K = 1024
SUB = 128


def _vq_kernel(x_ref, cb_ref, cbT_ref, out_ref):
    xs = x_ref[0, 0]
    cb = cb_ref[0]
    cbT = cbT_ref[0]
    m = jnp.dot(cb, xs, preferred_element_type=jnp.float32)
    cb2 = jnp.sum(cb * cb, axis=1, keepdims=True)
    d = cb2 - 2.0 * m
    idx = jnp.argmin(d, axis=0)
    onehot = (jax.lax.broadcasted_iota(jnp.int32, (K, T), 0)
              == idx[None, :]).astype(jnp.float32)
    out_ref[0, 0] = jnp.dot(cbT, onehot,
                            preferred_element_type=jnp.float32)


def kernel(x, codebooks):
    xr = x.reshape(B, G, SUB, T)
    cbT = codebooks.transpose(0, 2, 1)
    out = pl.pallas_call(
        _vq_kernel,
        grid=(G, B),
        in_specs=[
            pl.BlockSpec((1, 1, SUB, T), lambda g, b: (b, g, 0, 0)),
            pl.BlockSpec((1, K, SUB), lambda g, b: (g, 0, 0)),
            pl.BlockSpec((1, SUB, K), lambda g, b: (g, 0, 0)),
        ],
        out_specs=pl.BlockSpec((1, 1, SUB, T), lambda g, b: (b, g, 0, 0)),
        out_shape=jax.ShapeDtypeStruct((B, G, SUB, T), jnp.float32),
    )(xr, codebooks, cbT)
    out = out.reshape(B, C, F, T)
    return (out, out)
